# baseline (speedup 1.0000x reference)
import functools

import jax
import jax.numpy as jnp
from jax import lax
from jax.experimental import pallas as pl
from jax.experimental.pallas import tpu as pltpu
from jax.experimental.pallas import tpu_sc as plsc

THRESHOLD = 0.5
N, H, W = 5000, 64, 64
L = 16
NC, NS = 2, 16
NW = NC * NS
CHUNK = 8
NCHUNKS = N // CHUNK
ITERS = -(-NCHUNKS // NW)

_mesh = plsc.VectorSubcoreMesh(core_axis_name="c", subcore_axis_name="s")


@functools.partial(
    pl.kernel,
    mesh=_mesh,
    out_type=jax.ShapeDtypeStruct((N, L), jnp.float32),
    scratch_types=[
        pltpu.VMEM((CHUNK, H, W), jnp.float32),
        pltpu.VMEM((CHUNK, L), jnp.float32),
    ],
    compiler_params=pltpu.CompilerParams(needs_layout_passes=False),
)
def _boxes_sc(masks_hbm, out_hbm, buf, obuf):
    wid = lax.axis_index("s") * NC + lax.axis_index("c")
    lane_i = lax.iota(jnp.int32, L)
    lane_f = lane_i.astype(jnp.float32)
    neg_inf = jnp.full((L,), -jnp.inf, jnp.float32)
    h_full = jnp.full((L,), float(H), jnp.float32)
    neg1 = jnp.full((L,), -1.0, jnp.float32)

    def chunk_body(i, _):
        cid = i * NW + wid

        @pl.when(cid < NCHUNKS)
        def _process():
            base = cid * CHUNK
            pltpu.sync_copy(masks_hbm.at[pl.ds(base, CHUNK)], buf)

            for m in range(CHUNK):
                def row_body(r, carry):
                    c0, c1, c2, c3, mn, mx = carry
                    v0 = buf[m, r, pl.ds(0, L)]
                    v1 = buf[m, r, pl.ds(L, L)]
                    v2 = buf[m, r, pl.ds(2 * L, L)]
                    v3 = buf[m, r, pl.ds(3 * L, L)]
                    c0 = jnp.maximum(c0, v0)
                    c1 = jnp.maximum(c1, v1)
                    c2 = jnp.maximum(c2, v2)
                    c3 = jnp.maximum(c3, v3)
                    rmax = jnp.maximum(jnp.maximum(v0, v1),
                                       jnp.maximum(v2, v3))
                    has = rmax > THRESHOLD
                    rf = r.astype(jnp.float32)
                    mn = jnp.minimum(mn, jnp.where(has, rf, float(H)))
                    mx = jnp.maximum(mx, jnp.where(has, rf, -1.0))
                    return c0, c1, c2, c3, mn, mx

                c0, c1, c2, c3, mn, mx = lax.fori_loop(
                    0, H, row_body,
                    (neg_inf, neg_inf, neg_inf, neg_inf, h_full, neg1),
                    unroll=4)

                min_r = jnp.min(mn)
                max_r = jnp.max(mx)
                mnc = h_full
                mxc = neg1
                for j, cj in enumerate((c0, c1, c2, c3)):
                    chas = cj > THRESHOLD
                    cf = lane_f + float(j * L)
                    mnc = jnp.minimum(mnc, jnp.where(chas, cf, float(W)))
                    mxc = jnp.maximum(mxc, jnp.where(chas, cf, -1.0))
                min_c = jnp.min(mnc)
                max_c = jnp.max(mxc)

                res = jnp.where(
                    lane_i == 0, min_c,
                    jnp.where(lane_i == 1, min_r,
                              jnp.where(lane_i == 2, max_c, max_r)))
                obuf[m] = res

            pltpu.sync_copy(obuf, out_hbm.at[pl.ds(base, CHUNK)])

        return 0

    lax.fori_loop(0, ITERS, chunk_body, 0)


def kernel(masks):
    flat = _boxes_sc(masks)
    boxes_2d = flat[:, :4].reshape(N, 2, 2)
    return masks, boxes_2d


# speedup vs baseline: 3.7483x; 3.7483x over previous
import functools

import jax
import jax.numpy as jnp
from jax import lax
from jax.experimental import pallas as pl
from jax.experimental.pallas import tpu as pltpu
from jax.experimental.pallas import tpu_sc as plsc

THRESHOLD = 0.5
N, H, W = 5000, 64, 64
L = 16
NC, NS = 2, 16
NW = NC * NS
BN = 256
G = -(-N // BN)
NPAD = G * BN
CHL = 128
NCH = NPAD // CHL
SC_ITERS = -(-NCH // NW)
NG = CHL // L


def _tc_body(x_ref, cp_ref, rm_ref, cm_ref):
    x = x_ref[...]
    cp_ref[...] = x
    rm_ref[...] = jnp.max(x, axis=1)
    cm_ref[...] = jnp.max(x, axis=0)


_tc_pass = pl.pallas_call(
    _tc_body,
    grid=(G,),
    in_specs=[pl.BlockSpec((H, W, BN), lambda g: (0, 0, g))],
    out_specs=[
        pl.BlockSpec((H, W, BN), lambda g: (0, 0, g)),
        pl.BlockSpec((H, BN), lambda g: (0, g)),
        pl.BlockSpec((W, BN), lambda g: (0, g)),
    ],
    out_shape=[
        jax.ShapeDtypeStruct((H, W, N), jnp.float32),
        jax.ShapeDtypeStruct((H, NPAD), jnp.float32),
        jax.ShapeDtypeStruct((W, NPAD), jnp.float32),
    ],
)

_mesh = plsc.VectorSubcoreMesh(core_axis_name="c", subcore_axis_name="s")


@functools.partial(
    pl.kernel,
    mesh=_mesh,
    out_type=jax.ShapeDtypeStruct((4, NPAD), jnp.float32),
    scratch_types=[
        pltpu.VMEM((H, CHL), jnp.float32),
        pltpu.VMEM((W, CHL), jnp.float32),
        pltpu.VMEM((4, CHL), jnp.float32),
    ],
    compiler_params=pltpu.CompilerParams(needs_layout_passes=False),
)
def _sc_boxes(rm_hbm, cm_hbm, out_hbm, rbuf, cbuf, obuf):
    wid = lax.axis_index("s") * NC + lax.axis_index("c")
    h_full = jnp.full((L,), float(H), jnp.float32)
    neg1 = jnp.full((L,), -1.0, jnp.float32)

    for i in range(SC_ITERS):
        cid = i * NW + wid

        @pl.when(cid < NCH)
        def _process():
            base = cid * CHL
            pltpu.sync_copy(rm_hbm.at[:, pl.ds(base, CHL)], rbuf)
            pltpu.sync_copy(cm_hbm.at[:, pl.ds(base, CHL)], cbuf)

            for g in range(NG):
                def body(r, carry):
                    mnr, mxr, mnc, mxc = carry
                    vr = rbuf[r, pl.ds(g * L, L)]
                    vc = cbuf[r, pl.ds(g * L, L)]
                    rf = r.astype(jnp.float32)
                    br = vr > THRESHOLD
                    bc = vc > THRESHOLD
                    mnr = jnp.minimum(mnr, jnp.where(br, rf, float(H)))
                    mxr = jnp.maximum(mxr, jnp.where(br, rf, -1.0))
                    mnc = jnp.minimum(mnc, jnp.where(bc, rf, float(W)))
                    mxc = jnp.maximum(mxc, jnp.where(bc, rf, -1.0))
                    return mnr, mxr, mnc, mxc

                mnr, mxr, mnc, mxc = lax.fori_loop(
                    0, H, body, (h_full, neg1, h_full, neg1), unroll=8)

                obuf[0, pl.ds(g * L, L)] = mnc
                obuf[1, pl.ds(g * L, L)] = mnr
                obuf[2, pl.ds(g * L, L)] = mxc
                obuf[3, pl.ds(g * L, L)] = mxr

            pltpu.sync_copy(obuf, out_hbm.at[:, pl.ds(base, CHL)])


def kernel(masks):
    mt = jnp.transpose(masks, (1, 2, 0))
    cp, rm, cm = _tc_pass(mt)
    b4 = _sc_boxes(rm, cm)
    masks_out = jnp.transpose(cp, (2, 0, 1))
    boxes_2d = jnp.transpose(b4[:, :N]).reshape(N, 2, 2)
    return masks_out, boxes_2d
